# Initial kernel scaffold; baseline (speedup 1.0000x reference)
import jax
import jax.numpy as jnp
from jax import lax
from jax.experimental import pallas as pl
from jax.experimental.pallas import tpu as pltpu
from jax.experimental.pallas import tpu_sc as plsc

_N = 10000
_E = 160000
_IN = 256
_HID = 256
_HEADS = 4

_EPAD = 161280
_EPT = _EPAD // 32
_K = 80
_ROWS2D = _EPAD // _K
_RPT = _ROWS2D // 16
_NPAD = 10240

_f32 = jnp.float32
_i32 = jnp.int32


def _make_edge_kernel(H):
    mesh = plsc.VectorSubcoreMesh(core_axis_name="c", subcore_axis_name="s")

    def body(alog, srch, dsth, p_out, den_out,
             asrcb, adstb, srcb, dstb, pb, part, rbuf, dsumb, stage):
        c = lax.axis_index("c")
        s = lax.axis_index("s")
        w = c * 16 + s
        base = w * _EPT
        pltpu.sync_copy(srch.at[pl.ds(base, _EPT)], srcb)
        pltpu.sync_copy(dsth.at[pl.ds(base, _EPT)], dstb)

        for h in range(H):
            pltpu.sync_copy(alog.at[h], asrcb)
            pltpu.sync_copy(alog.at[H + h], adstb)

            def zbody(i, _):
                part[pl.ds(i * 16, 16)] = jnp.zeros((16,), _f32)
                return 0
            lax.fori_loop(0, _NPAD // 16, zbody, 0)

            def ebody(i, _):
                off = i * 16
                sv = srcb[pl.ds(off, 16)]
                dv = dstb[pl.ds(off, 16)]
                asv = plsc.load_gather(asrcb, [sv])
                adv = plsc.load_gather(adstb, [dv])
                e = asv + adv
                e = jnp.where(e >= 0.0, e, 0.2 * e)
                p = jnp.exp(e)
                eid = base + off + lax.broadcasted_iota(_i32, (16,), 0)
                p = jnp.where(eid < _E, p, 0.0)
                pb[pl.ds(off, 16)] = p
                plsc.addupdate_scatter(part, [dv], p)
                return 0
            lax.fori_loop(0, _EPT // 16, ebody, 0)

            pltpu.sync_copy(pb, p_out.at[h, pl.ds(base, _EPT)])

            pltpu.sync_copy(part, stage.at[s])
            plsc.subcore_barrier()
            colbase = s * (_NPAD // 16)
            pltpu.sync_copy(stage.at[:, pl.ds(colbase, _NPAD // 16)], rbuf)

            def rbody(j, _):
                acc = rbuf[0, pl.ds(j * 16, 16)]
                for r in range(1, 16):
                    acc = acc + rbuf[r, pl.ds(j * 16, 16)]
                dsumb[pl.ds(j * 16, 16)] = acc
                return 0
            lax.fori_loop(0, _NPAD // 256, rbody, 0)
            pltpu.sync_copy(dsumb, den_out.at[c, h, pl.ds(colbase, _NPAD // 16)])
            plsc.subcore_barrier()

    return pl.kernel(
        body,
        out_type=(
            jax.ShapeDtypeStruct((H, _EPAD), _f32),
            jax.ShapeDtypeStruct((2, H, _NPAD), _f32),
        ),
        mesh=mesh,
        scratch_types=[
            pltpu.VMEM((_N,), _f32),
            pltpu.VMEM((_N,), _f32),
            pltpu.VMEM((_EPT,), _i32),
            pltpu.VMEM((_EPT,), _i32),
            pltpu.VMEM((_EPT,), _f32),
            pltpu.VMEM((_NPAD,), _f32),
            pltpu.VMEM((16, _NPAD // 16), _f32),
            pltpu.VMEM((_NPAD // 16,), _f32),
            pltpu.VMEM_SHARED((16, _NPAD), _f32),
        ],
    )


def _make_msg_kernel(NCH, H):
    CPC = NCH // 2
    HB = NCH // H
    mesh = plsc.VectorSubcoreMesh(core_axis_name="c", subcore_axis_name="s")

    def body(h_hbm, src2d, dst2d, p3, out_hbm,
             srcb, dstb, pbuf, gbuf, zbuf, outsp):
        c = lax.axis_index("c")
        s = lax.axis_index("s")
        rbase = s * _RPT
        pltpu.sync_copy(src2d.at[pl.ds(rbase, _RPT)], srcb)
        pltpu.sync_copy(dst2d.at[pl.ds(rbase, _RPT)], dstb)

        def zb(i, _):
            for k8 in range(8):
                zbuf[i, pl.ds(k8 * 16, 16)] = jnp.zeros((16,), _f32)
            return 0
        lax.fori_loop(0, 125, zb, 0)

        for q in range(CPC):
            for z in range(5):
                pltpu.sync_copy(zbuf, outsp.at[pl.ds(s * 625 + z * 125, 125)])
            plsc.subcore_barrier()

            h0 = (0 * CPC + q) // HB
            h1 = (1 * CPC + q) // HB
            if h0 == h1:
                pltpu.sync_copy(p3.at[h0, pl.ds(rbase, _RPT)], pbuf)
            else:
                @pl.when(c == 0)
                def _():
                    pltpu.sync_copy(p3.at[h0, pl.ds(rbase, _RPT)], pbuf)

                @pl.when(c == 1)
                def _():
                    pltpu.sync_copy(p3.at[h1, pl.ds(rbase, _RPT)], pbuf)

            chv = c * CPC + q

            def step(j, _):
                pltpu.sync_copy(h_hbm.at[chv].at[srcb.at[j]], gbuf)

                def scale_e(e, _):
                    pe = pbuf[j, e]
                    for k8 in range(8):
                        sl = pl.ds(k8 * 16, 16)
                        gbuf[e, sl] = gbuf[e, sl] * pe
                    return 0
                lax.fori_loop(0, _K, scale_e, 0)
                pltpu.sync_copy(gbuf, outsp.at[dstb.at[j]], add=True)
                return 0
            lax.fori_loop(0, _RPT, step, 0)
            plsc.subcore_barrier()
            for z in range(5):
                rr = s * 625 + z * 125
                pltpu.sync_copy(outsp.at[pl.ds(rr, 125)],
                                out_hbm.at[chv, pl.ds(rr, 125)])
            plsc.subcore_barrier()

    return pl.kernel(
        body,
        out_type=jax.ShapeDtypeStruct((NCH, _N, 128), _f32),
        mesh=mesh,
        scratch_types=[
            pltpu.VMEM((_RPT, _K), _i32),
            pltpu.VMEM((_RPT, _K), _i32),
            pltpu.VMEM((_RPT, _K), _f32),
            pltpu.VMEM((_K, 128), _f32),
            pltpu.VMEM((125, 128), _f32),
            pltpu.VMEM_SHARED((_N, 128), _f32),
        ],
    )


def _mm_a(x, W1, S1):
    def body(x_ref, w_ref, s_ref, h_ref, al_ref):
        c = pl.program_id(1)
        hb = jnp.dot(x_ref[...], w_ref[...], preferred_element_type=_f32)
        h_ref[0] = hb
        ab = jnp.dot(hb, s_ref[0], preferred_element_type=_f32)

        @pl.when(c == 0)
        def _():
            al_ref[...] = ab

        @pl.when(c != 0)
        def _():
            al_ref[...] += ab

    return pl.pallas_call(
        body,
        grid=(20, 8),
        in_specs=[
            pl.BlockSpec((500, _IN), lambda nb, c: (nb, 0)),
            pl.BlockSpec((_IN, 128), lambda nb, c: (0, c)),
            pl.BlockSpec((1, 128, 128), lambda nb, c: (c, 0, 0)),
        ],
        out_specs=[
            pl.BlockSpec((1, 500, 128), lambda nb, c: (c, nb, 0)),
            pl.BlockSpec((500, 128), lambda nb, c: (nb, 0)),
        ],
        out_shape=[
            jax.ShapeDtypeStruct((8, _N, 128), _f32),
            jax.ShapeDtypeStruct((_N, 128), _f32),
        ],
    )(x, W1, S1)


def _mm_b(msg1, den1, b1r, W2r, S2):
    def body(m_ref, d_ref, b_ref, w_ref, s_ref, h2_ref, al_ref):
        c = pl.program_id(1)
        k = pl.program_id(2)
        den = d_ref[0, 0, :] + d_ref[1, 0, :] + 1e-16
        x2 = m_ref[0] / den[:, None] + b_ref[0][None, :]
        x2 = jnp.maximum(x2, 0.0)
        part = jnp.dot(x2, w_ref[0, :, 0, :], preferred_element_type=_f32)

        @pl.when(k == 0)
        def _():
            h2_ref[0] = part

        @pl.when(k != 0)
        def _():
            h2_ref[0] += part

        @pl.when(jnp.logical_and(k == 7, c == 0))
        def _():
            al_ref[...] = jnp.dot(h2_ref[0], s_ref[0],
                                  preferred_element_type=_f32)

        @pl.when(jnp.logical_and(k == 7, c == 1))
        def _():
            al_ref[...] += jnp.dot(h2_ref[0], s_ref[0],
                                   preferred_element_type=_f32)

    return pl.pallas_call(
        body,
        grid=(20, 2, 8),
        in_specs=[
            pl.BlockSpec((1, 500, 128), lambda nb, c, k: (k, nb, 0)),
            pl.BlockSpec((2, 1, 500), lambda nb, c, k: (0, k // 2, nb)),
            pl.BlockSpec((1, 128), lambda nb, c, k: (k, 0)),
            pl.BlockSpec((1, 128, 1, 128), lambda nb, c, k: (k, 0, c, 0)),
            pl.BlockSpec((1, 128, 128), lambda nb, c, k: (c, 0, 0)),
        ],
        out_specs=[
            pl.BlockSpec((1, 500, 128), lambda nb, c, k: (c, nb, 0)),
            pl.BlockSpec((500, 128), lambda nb, c, k: (nb, 0)),
        ],
        out_shape=[
            jax.ShapeDtypeStruct((2, _N, 128), _f32),
            jax.ShapeDtypeStruct((_N, 128), _f32),
        ],
    )(msg1, den1, b1r, W2r, S2)


def _mm_c(msg2, den2, b2r):
    def body(m_ref, d_ref, b_ref, o_ref):
        den = d_ref[0, 0, :] + d_ref[1, 0, :] + 1e-16
        o_ref[...] = m_ref[0] / den[:, None] + b_ref[0][None, :]

    return pl.pallas_call(
        body,
        grid=(20, 2),
        in_specs=[
            pl.BlockSpec((1, 500, 128), lambda nb, c: (c, nb, 0)),
            pl.BlockSpec((2, 1, 500), lambda nb, c: (0, 0, nb)),
            pl.BlockSpec((1, 128), lambda nb, c: (c, 0)),
        ],
        out_specs=pl.BlockSpec((500, 128), lambda nb, c: (nb, c)),
        out_shape=jax.ShapeDtypeStruct((_N, _IN), _f32),
    )(msg2, den2, b2r)


def kernel(x, edge_index, W1, a1_src, a1_dst, b1, W2, a2_src, a2_dst, b2):
    S1 = jnp.zeros((8, 128, 128), _f32)
    for c in range(8):
        h = c // 2
        off = (c % 2) * 128
        S1 = S1.at[c, :, h].set(a1_src[h, off:off + 128])
        S1 = S1.at[c, :, _HEADS + h].set(a1_dst[h, off:off + 128])
    S2 = jnp.zeros((2, 128, 128), _f32)
    for c in range(2):
        S2 = S2.at[c, :, 0].set(a2_src[0, c * 128:(c + 1) * 128])
        S2 = S2.at[c, :, 1].set(a2_dst[0, c * 128:(c + 1) * 128])
    W2r = W2.reshape(8, 128, 2, 128)
    b1r = b1.reshape(8, 128)
    b2r = b2.reshape(2, 128)

    pad = jnp.zeros((_EPAD - _E,), _i32)
    src_p = jnp.concatenate([edge_index[0], pad])
    dst_p = jnp.concatenate([edge_index[1], pad])
    src2d = src_p.reshape(_ROWS2D, _K)
    dst2d = dst_p.reshape(_ROWS2D, _K)

    edge1 = _make_edge_kernel(_HEADS)
    edge2 = _make_edge_kernel(1)
    msg_k1 = _make_msg_kernel(8, _HEADS)
    msg_k2 = _make_msg_kernel(2, 1)

    h1, al1 = _mm_a(x, W1, S1)
    alog1 = jnp.transpose(al1[:, :2 * _HEADS])
    p1, den1 = edge1(alog1, src_p, dst_p)
    msg1 = msg_k1(h1, src2d, dst2d, p1.reshape(_HEADS, _ROWS2D, _K))

    h2, al2 = _mm_b(msg1, den1, b1r, W2r, S2)
    alog2 = jnp.transpose(al2[:, :2])
    p2, den2 = edge2(alog2, src_p, dst_p)
    msg2 = msg_k2(h2, src2d, dst2d, p2.reshape(1, _ROWS2D, _K))

    return _mm_c(msg2, den2, b2r)


# speedup vs baseline: 10.2077x; 10.2077x over previous
import jax
import jax.numpy as jnp
from jax import lax
from jax.experimental import pallas as pl
from jax.experimental.pallas import tpu as pltpu
from jax.experimental.pallas import tpu_sc as plsc

_N = 10000
_E = 160000
_IN = 256
_HID = 256
_HEADS = 4

_EPAD = 163840
_EPT = _EPAD // 32
_K = 80
_ROWS2D = _EPAD // _K
_RPT = _ROWS2D // 16
_NPAD = 10240

_f32 = jnp.float32
_i32 = jnp.int32


def _make_edge_kernel(H):
    mesh = plsc.VectorSubcoreMesh(core_axis_name="c", subcore_axis_name="s")

    NC16 = _NPAD // 16

    def body(alog, srch, dsth, p_out, den_out,
             asrcb, adstb, srcb, dstb, pb, part, rbuf, dsumb, stage):
        c = lax.axis_index("c")
        s = lax.axis_index("s")
        w = c * 16 + s
        base = w * _EPT
        pltpu.sync_copy(srch.at[pl.ds(base, _EPT)], srcb)
        pltpu.sync_copy(dsth.at[pl.ds(base, _EPT)], dstb)

        for h in range(H):
            pltpu.sync_copy(alog.at[pl.ds(h * _NPAD, _NPAD)], asrcb)
            pltpu.sync_copy(alog.at[pl.ds((H + h) * _NPAD, _NPAD)], adstb)

            def zbody(i, _):
                part[pl.ds(i * 16, 16)] = jnp.zeros((16,), _f32)
                return 0
            lax.fori_loop(0, _NPAD // 16, zbody, 0)

            def ebody(i, _):
                off = i * 16
                sv = srcb[pl.ds(off, 16)]
                dv = dstb[pl.ds(off, 16)]
                asv = plsc.load_gather(asrcb, [sv])
                adv = plsc.load_gather(adstb, [dv])
                e = asv + adv
                e = jnp.where(e >= 0.0, e, 0.2 * e)
                p = jnp.exp(e)
                eid = base + off + lax.broadcasted_iota(_i32, (16,), 0)
                p = jnp.where(eid < _E, p, 0.0)
                pb[pl.ds(off, 16)] = p
                plsc.addupdate_scatter(part, [dv], p)
                return 0
            lax.fori_loop(0, _EPT // 16, ebody, 0)

            pltpu.sync_copy(pb, p_out.at[pl.ds(h * _EPAD + base, _EPT)])

            pltpu.sync_copy(part, stage.at[pl.ds(s * _NPAD, _NPAD)])
            plsc.subcore_barrier()
            colbase = s * NC16
            for r in range(16):
                pltpu.sync_copy(stage.at[pl.ds(r * _NPAD + colbase, NC16)],
                                rbuf.at[pl.ds(r * NC16, NC16)])

            def rbody(j, _):
                acc = rbuf[pl.ds(j * 16, 16)]
                for r in range(1, 16):
                    acc = acc + rbuf[pl.ds(r * NC16 + j * 16, 16)]
                dsumb[pl.ds(j * 16, 16)] = acc
                return 0
            lax.fori_loop(0, NC16 // 16, rbody, 0)
            pltpu.sync_copy(
                dsumb,
                den_out.at[pl.ds((c * H + h) * _NPAD + colbase, NC16)])
            plsc.subcore_barrier()

    return pl.kernel(
        body,
        out_type=(
            jax.ShapeDtypeStruct((H * _EPAD,), _f32),
            jax.ShapeDtypeStruct((2 * H * _NPAD,), _f32),
        ),
        mesh=mesh,
        compiler_params=pltpu.CompilerParams(needs_layout_passes=False),
        scratch_types=[
            pltpu.VMEM((_NPAD,), _f32),
            pltpu.VMEM((_NPAD,), _f32),
            pltpu.VMEM((_EPT,), _i32),
            pltpu.VMEM((_EPT,), _i32),
            pltpu.VMEM((_EPT,), _f32),
            pltpu.VMEM((_NPAD,), _f32),
            pltpu.VMEM((16 * NC16,), _f32),
            pltpu.VMEM((NC16,), _f32),
            pltpu.VMEM_SHARED((16 * _NPAD,), _f32),
        ],
    )


def _make_msg_kernel(NCH, H):
    CPC = NCH // 2
    HB = NCH // H
    mesh = plsc.VectorSubcoreMesh(core_axis_name="c", subcore_axis_name="s")

    EPS = _EPAD // 16

    def body(h_hbm, srcf, dst2d, pf, out_hbm,
             srcb, dstb, pbuf, gbuf, zbuf, outsp):
        c = lax.axis_index("c")
        s = lax.axis_index("s")
        ebase = s * EPS
        pltpu.sync_copy(srcf.at[pl.ds(ebase, EPS)], srcb)
        pltpu.sync_copy(dst2d.at[pl.ds(s * _RPT, _RPT)], dstb)

        def zb(i, _):
            for k8 in range(8):
                zbuf[i, pl.ds(k8 * 16, 16)] = jnp.zeros((16,), _f32)
            return 0
        lax.fori_loop(0, 16, zb, 0)

        rowbase = s * 640

        for q in range(CPC):
            def zs(z, _):
                pltpu.sync_copy(zbuf, outsp.at[pl.ds(rowbase + z * 16, 16)])
                return 0
            lax.fori_loop(0, 40, zs, 0)
            plsc.subcore_barrier()

            h0 = (0 * CPC + q) // HB
            h1 = (1 * CPC + q) // HB
            if h0 == h1:
                pltpu.sync_copy(pf.at[pl.ds(h0 * _EPAD + ebase, EPS)], pbuf)
            else:
                @pl.when(c == 0)
                def _():
                    pltpu.sync_copy(pf.at[pl.ds(h0 * _EPAD + ebase, EPS)],
                                    pbuf)

                @pl.when(c == 1)
                def _():
                    pltpu.sync_copy(pf.at[pl.ds(h1 * _EPAD + ebase, EPS)],
                                    pbuf)

            chv = c * CPC + q

            def step(j, _):
                pltpu.sync_copy(h_hbm.at[chv].at[srcb.at[pl.ds(j * _K, _K)]],
                                gbuf)

                def scale_g(g, _):
                    pv = pbuf[pl.ds(j * _K + g * 16, 16)]
                    for ee in range(16):
                        pe = pv[ee]
                        e = g * 16 + ee
                        for k8 in range(8):
                            sl = pl.ds(k8 * 16, 16)
                            gbuf[e, sl] = gbuf[e, sl] * pe
                    return 0
                lax.fori_loop(0, _K // 16, scale_g, 0)
                pltpu.sync_copy(gbuf, outsp.at[dstb.at[j]], add=True)
                return 0
            lax.fori_loop(0, _RPT, step, 0)
            plsc.subcore_barrier()
            pltpu.sync_copy(outsp.at[pl.ds(rowbase, 640)],
                            out_hbm.at[chv, pl.ds(rowbase, 640)])
            plsc.subcore_barrier()

    return pl.kernel(
        body,
        out_type=jax.ShapeDtypeStruct((NCH, _NPAD, 128), _f32),
        mesh=mesh,
        compiler_params=pltpu.CompilerParams(needs_layout_passes=False),
        scratch_types=[
            pltpu.VMEM((EPS,), _i32),
            pltpu.VMEM((_RPT, _K), _i32),
            pltpu.VMEM((EPS,), _f32),
            pltpu.VMEM((_K, 128), _f32),
            pltpu.VMEM((16, 128), _f32),
            pltpu.VMEM_SHARED((_NPAD, 128), _f32),
        ],
    )


def _mm_a(x, W1, S1):
    def body(x_ref, w_ref, s_ref, h_ref, al_ref):
        c = pl.program_id(1)
        hb = jnp.dot(x_ref[...], w_ref[...], preferred_element_type=_f32)
        h_ref[0] = hb
        ab = jnp.dot(hb, s_ref[0], preferred_element_type=_f32)

        @pl.when(c == 0)
        def _():
            al_ref[...] = ab

        @pl.when(c != 0)
        def _():
            al_ref[...] += ab

    return pl.pallas_call(
        body,
        grid=(10, 8),
        in_specs=[
            pl.BlockSpec((1024, _IN), lambda nb, c: (nb, 0)),
            pl.BlockSpec((_IN, 128), lambda nb, c: (0, c)),
            pl.BlockSpec((1, 128, 128), lambda nb, c: (c, 0, 0)),
        ],
        out_specs=[
            pl.BlockSpec((1, 1024, 128), lambda nb, c: (c, nb, 0)),
            pl.BlockSpec((1024, 128), lambda nb, c: (nb, 0)),
        ],
        out_shape=[
            jax.ShapeDtypeStruct((8, _NPAD, 128), _f32),
            jax.ShapeDtypeStruct((_NPAD, 128), _f32),
        ],
    )(x, W1, S1)


def _mm_b(msg1, den1, b1r, W2r, S2):
    def body(m_ref, d_ref, b_ref, w_ref, s_ref, h2_ref, al_ref):
        c = pl.program_id(1)
        k = pl.program_id(2)
        den = d_ref[0, 0] + d_ref[1, 0] + 1e-16
        x2v = m_ref[0].reshape(8, 128, 128) / den[:, :, None]
        x2 = x2v.reshape(1024, 128) + b_ref[0, 0][None, :]
        x2 = jnp.maximum(x2, 0.0)
        part = jnp.dot(x2, w_ref[0, 0], preferred_element_type=_f32)

        @pl.when(k == 0)
        def _():
            h2_ref[0] = part

        @pl.when(k != 0)
        def _():
            h2_ref[0] += part

        @pl.when(jnp.logical_and(k == 7, c == 0))
        def _():
            al_ref[...] = jnp.dot(h2_ref[0], s_ref[0],
                                  preferred_element_type=_f32)

        @pl.when(jnp.logical_and(k == 7, c == 1))
        def _():
            al_ref[...] += jnp.dot(h2_ref[0], s_ref[0],
                                   preferred_element_type=_f32)

    return pl.pallas_call(
        body,
        grid=(10, 2, 8),
        in_specs=[
            pl.BlockSpec((1, 1024, 128), lambda nb, c, k: (k, nb, 0)),
            pl.BlockSpec((2, 1, 8, 128), lambda nb, c, k: (0, k, nb, 0)),
            pl.BlockSpec((1, 1, 128), lambda nb, c, k: (k, 0, 0)),
            pl.BlockSpec((1, 1, 128, 128), lambda nb, c, k: (c, k, 0, 0)),
            pl.BlockSpec((1, 128, 128), lambda nb, c, k: (c, 0, 0)),
        ],
        out_specs=[
            pl.BlockSpec((1, 1024, 128), lambda nb, c, k: (c, nb, 0)),
            pl.BlockSpec((1024, 128), lambda nb, c, k: (nb, 0)),
        ],
        out_shape=[
            jax.ShapeDtypeStruct((2, _NPAD, 128), _f32),
            jax.ShapeDtypeStruct((_NPAD, 128), _f32),
        ],
    )(msg1, den1, b1r, W2r, S2)


def _mm_c(msg2, den2, b2r):
    def body(m_ref, d_ref, b_ref, o_ref):
        den = d_ref[0, 0] + d_ref[1, 0] + 1e-16
        ov = m_ref[0].reshape(8, 128, 128) / den[:, :, None]
        o_ref[...] = ov.reshape(1024, 128) + b_ref[0, 0][None, :]

    return pl.pallas_call(
        body,
        grid=(10, 2),
        in_specs=[
            pl.BlockSpec((1, 1024, 128), lambda nb, c: (c, nb, 0)),
            pl.BlockSpec((2, 1, 8, 128), lambda nb, c: (0, 0, nb, 0)),
            pl.BlockSpec((1, 1, 128), lambda nb, c: (c, 0, 0)),
        ],
        out_specs=pl.BlockSpec((1024, 128), lambda nb, c: (nb, c)),
        out_shape=jax.ShapeDtypeStruct((_NPAD, _IN), _f32),
    )(msg2, den2, b2r)


def kernel(x, edge_index, W1, a1_src, a1_dst, b1, W2, a2_src, a2_dst, b2):
    S1 = jnp.zeros((8, 128, 128), _f32)
    for c in range(8):
        h = c // 2
        off = (c % 2) * 128
        S1 = S1.at[c, :, h].set(a1_src[h, off:off + 128])
        S1 = S1.at[c, :, _HEADS + h].set(a1_dst[h, off:off + 128])
    S2 = jnp.zeros((2, 128, 128), _f32)
    for c in range(2):
        S2 = S2.at[c, :, 0].set(a2_src[0, c * 128:(c + 1) * 128])
        S2 = S2.at[c, :, 1].set(a2_dst[0, c * 128:(c + 1) * 128])
    W2r = W2.reshape(8, 128, 2, 128).transpose(2, 0, 1, 3)
    b1r = b1.reshape(8, 1, 128)
    b2r = b2.reshape(2, 1, 128)
    x_pad = jnp.pad(x, ((0, _NPAD - _N), (0, 0)))

    pad = jnp.zeros((_EPAD - _E,), _i32)
    src_p = jnp.concatenate([edge_index[0], pad])
    dst_p = jnp.concatenate([edge_index[1], pad])
    dst2d = dst_p.reshape(_ROWS2D, _K)

    edge1 = _make_edge_kernel(_HEADS)
    edge2 = _make_edge_kernel(1)
    msg_k1 = _make_msg_kernel(8, _HEADS)
    msg_k2 = _make_msg_kernel(2, 1)

    h1, al1 = _mm_a(x_pad, W1, S1)
    alog1 = jnp.transpose(al1[:, :2 * _HEADS]).reshape(-1)
    p1, den1 = edge1(alog1, src_p, dst_p)
    msg1 = msg_k1(h1, src_p, dst2d, p1)

    den1k = den1.reshape(2, _HEADS, 80, 128)[:, jnp.array([0, 0, 1, 1, 2, 2, 3, 3])]
    h2, al2 = _mm_b(msg1, den1k, b1r, W2r, S2)

    alog2 = jnp.transpose(al2[:, :2]).reshape(-1)
    p2, den2 = edge2(alog2, src_p, dst_p)
    msg2 = msg_k2(h2, src_p, dst2d, p2)

    return _mm_c(msg2, den2.reshape(2, 1, 80, 128), b2r)[:_N]


# speedup vs baseline: 13.2632x; 1.2993x over previous
import jax
import jax.numpy as jnp
from jax import lax
from jax.experimental import pallas as pl
from jax.experimental.pallas import tpu as pltpu
from jax.experimental.pallas import tpu_sc as plsc

_N = 10000
_E = 160000
_IN = 256
_HID = 256
_HEADS = 4

_EPAD = 163840
_EPT = _EPAD // 32
_K = 80
_ROWS2D = _EPAD // _K
_RPT = _ROWS2D // 16
_NPAD = 10240

_f32 = jnp.float32
_i32 = jnp.int32


def _make_edge_kernel(H):
    mesh = plsc.VectorSubcoreMesh(core_axis_name="c", subcore_axis_name="s")

    NC16 = _NPAD // 16

    def body(alog, srch, dsth, p_out, den_out,
             asrcb, adstb, srcb, dstb, pb, part, rbuf, dsumb, stage):
        c = lax.axis_index("c")
        s = lax.axis_index("s")
        w = c * 16 + s
        base = w * _EPT
        pltpu.sync_copy(srch.at[pl.ds(base, _EPT)], srcb)
        pltpu.sync_copy(dsth.at[pl.ds(base, _EPT)], dstb)

        for h in range(H):
            pltpu.sync_copy(alog.at[pl.ds(h * _NPAD, _NPAD)], asrcb)
            pltpu.sync_copy(alog.at[pl.ds((H + h) * _NPAD, _NPAD)], adstb)

            def zbody(i, _):
                part[pl.ds(i * 16, 16)] = jnp.zeros((16,), _f32)
                return 0
            lax.fori_loop(0, _NPAD // 16, zbody, 0)

            def ebody(i, _):
                off = i * 16
                sv = srcb[pl.ds(off, 16)]
                dv = dstb[pl.ds(off, 16)]
                asv = plsc.load_gather(asrcb, [sv])
                adv = plsc.load_gather(adstb, [dv])
                e = asv + adv
                e = jnp.where(e >= 0.0, e, 0.2 * e)
                p = jnp.exp(e)
                eid = base + off + lax.broadcasted_iota(_i32, (16,), 0)
                p = jnp.where(eid < _E, p, 0.0)
                pb[pl.ds(off, 16)] = p
                plsc.addupdate_scatter(part, [dv], p)
                return 0
            lax.fori_loop(0, _EPT // 16, ebody, 0)

            pltpu.sync_copy(pb, p_out.at[pl.ds(h * _EPAD + base, _EPT)])

            pltpu.sync_copy(part, stage.at[pl.ds(s * _NPAD, _NPAD)])
            plsc.subcore_barrier()
            colbase = s * NC16
            for r in range(16):
                pltpu.sync_copy(stage.at[pl.ds(r * _NPAD + colbase, NC16)],
                                rbuf.at[pl.ds(r * NC16, NC16)])

            def rbody(j, _):
                acc = rbuf[pl.ds(j * 16, 16)]
                for r in range(1, 16):
                    acc = acc + rbuf[pl.ds(r * NC16 + j * 16, 16)]
                dsumb[pl.ds(j * 16, 16)] = acc
                return 0
            lax.fori_loop(0, NC16 // 16, rbody, 0)
            pltpu.sync_copy(
                dsumb,
                den_out.at[pl.ds((c * H + h) * _NPAD + colbase, NC16)])
            plsc.subcore_barrier()

    return pl.kernel(
        body,
        out_type=(
            jax.ShapeDtypeStruct((H * _EPAD,), _f32),
            jax.ShapeDtypeStruct((2 * H * _NPAD,), _f32),
        ),
        mesh=mesh,
        compiler_params=pltpu.CompilerParams(needs_layout_passes=False),
        scratch_types=[
            pltpu.VMEM((_NPAD,), _f32),
            pltpu.VMEM((_NPAD,), _f32),
            pltpu.VMEM((_EPT,), _i32),
            pltpu.VMEM((_EPT,), _i32),
            pltpu.VMEM((_EPT,), _f32),
            pltpu.VMEM((_NPAD,), _f32),
            pltpu.VMEM((16 * NC16,), _f32),
            pltpu.VMEM((NC16,), _f32),
            pltpu.VMEM_SHARED((16 * _NPAD,), _f32),
        ],
    )


def _make_msg_kernel(NCH, H):
    CPC = NCH // 2
    HB = NCH // H
    mesh = plsc.VectorSubcoreMesh(core_axis_name="c", subcore_axis_name="s")

    EPS = _EPAD // 16

    HSTEP = _RPT // 2
    HEDGE = EPS // 2

    def body(h_hbm, srcf, dst2d, pf, out_hbm,
             srcb, dstb, pbuf, gbufa, gbufb, zbuf, outsp, gsema, gsemb):
        c = lax.axis_index("c")
        s = lax.axis_index("s")
        ebase = s * EPS

        def zb(i, _):
            for k8 in range(8):
                zbuf[i, pl.ds(k8 * 16, 16)] = jnp.zeros((16,), _f32)
            return 0
        lax.fori_loop(0, 16, zb, 0)

        rowbase = s * 640

        for q in range(CPC):
            def zs(z, _):
                pltpu.sync_copy(zbuf, outsp.at[pl.ds(rowbase + z * 16, 16)])
                return 0
            lax.fori_loop(0, 40, zs, 0)
            plsc.subcore_barrier()

            h0 = (0 * CPC + q) // HB
            h1 = (1 * CPC + q) // HB
            if h0 == h1:
                pltpu.sync_copy(pf.at[pl.ds(h0 * _EPAD + ebase, EPS)], pbuf)
            else:
                @pl.when(c == 0)
                def _():
                    pltpu.sync_copy(pf.at[pl.ds(h0 * _EPAD + ebase, EPS)],
                                    pbuf)

                @pl.when(c == 1)
                def _():
                    pltpu.sync_copy(pf.at[pl.ds(h1 * _EPAD + ebase, EPS)],
                                    pbuf)

            chv = c * CPC + q

            for half in range(2):
                pltpu.sync_copy(
                    srcf.at[pl.ds(ebase + half * HEDGE, HEDGE)], srcb)
                pltpu.sync_copy(
                    dst2d.at[pl.ds(s * _RPT + half * HSTEP, HSTEP)], dstb)

                def fire_gather(j, gb, gsem):
                    pltpu.async_copy(
                        h_hbm.at[chv].at[srcb.at[pl.ds(j * _K, _K)]],
                        gb, gsem)

                fire_gather(0, gbufa, gsema)
                fire_gather(1, gbufb, gsemb)

                poff = half * HEDGE

                def slot(j, gb, gsem):
                    pltpu.make_async_copy(h_hbm.at[chv, pl.ds(0, _K)], gb,
                                          gsem).wait()

                    def scale_g(g, _):
                        pv = pbuf[pl.ds(poff + j * _K + g * 16, 16)]
                        for ee in range(16):
                            pe = pv[ee]
                            e = g * 16 + ee
                            for k8 in range(8):
                                sl = pl.ds(k8 * 16, 16)
                                gb[e, sl] = gb[e, sl] * pe
                        return 0
                    lax.fori_loop(0, _K // 16, scale_g, 0)

                    pltpu.sync_copy(gb, outsp.at[dstb.at[j]], add=True)

                    @pl.when(j + 2 < HSTEP)
                    def _():
                        fire_gather(j + 2, gb, gsem)

                def pipe(i, _):
                    slot(2 * i, gbufa, gsema)
                    slot(2 * i + 1, gbufb, gsemb)
                    return 0
                lax.fori_loop(0, HSTEP // 2, pipe, 0)
            plsc.subcore_barrier()
            pltpu.sync_copy(outsp.at[pl.ds(rowbase, 640)],
                            out_hbm.at[chv, pl.ds(rowbase, 640)])
            plsc.subcore_barrier()

    return pl.kernel(
        body,
        out_type=jax.ShapeDtypeStruct((NCH, _NPAD, 128), _f32),
        mesh=mesh,
        compiler_params=pltpu.CompilerParams(needs_layout_passes=False),
        scratch_types=[
            pltpu.VMEM((EPS // 2,), _i32),
            pltpu.VMEM((_RPT // 2, _K), _i32),
            pltpu.VMEM((EPS,), _f32),
            pltpu.VMEM((_K, 128), _f32),
            pltpu.VMEM((_K, 128), _f32),
            pltpu.VMEM((16, 128), _f32),
            pltpu.VMEM_SHARED((_NPAD, 128), _f32),
            pltpu.SemaphoreType.DMA,
            pltpu.SemaphoreType.DMA,
        ],
    )


def _mm_a(x, W1, S1):
    def body(x_ref, w_ref, s_ref, h_ref, al_ref):
        c = pl.program_id(1)
        hb = jnp.dot(x_ref[...], w_ref[...], preferred_element_type=_f32)
        h_ref[0] = hb
        ab = jnp.dot(hb, s_ref[0], preferred_element_type=_f32)

        @pl.when(c == 0)
        def _():
            al_ref[...] = ab

        @pl.when(c != 0)
        def _():
            al_ref[...] += ab

    return pl.pallas_call(
        body,
        grid=(10, 8),
        in_specs=[
            pl.BlockSpec((1024, _IN), lambda nb, c: (nb, 0)),
            pl.BlockSpec((_IN, 128), lambda nb, c: (0, c)),
            pl.BlockSpec((1, 128, 128), lambda nb, c: (c, 0, 0)),
        ],
        out_specs=[
            pl.BlockSpec((1, 1024, 128), lambda nb, c: (c, nb, 0)),
            pl.BlockSpec((1024, 128), lambda nb, c: (nb, 0)),
        ],
        out_shape=[
            jax.ShapeDtypeStruct((8, _NPAD, 128), _f32),
            jax.ShapeDtypeStruct((_NPAD, 128), _f32),
        ],
    )(x, W1, S1)


def _mm_b(msg1, den1, b1r, W2r, S2):
    def body(m_ref, d_ref, b_ref, w_ref, s_ref, h2_ref, al_ref):
        c = pl.program_id(1)
        k = pl.program_id(2)
        den = d_ref[0, 0] + d_ref[1, 0] + 1e-16
        x2v = m_ref[0].reshape(8, 128, 128) / den[:, :, None]
        x2 = x2v.reshape(1024, 128) + b_ref[0, 0][None, :]
        x2 = jnp.maximum(x2, 0.0)
        part = jnp.dot(x2, w_ref[0, 0], preferred_element_type=_f32)

        @pl.when(k == 0)
        def _():
            h2_ref[0] = part

        @pl.when(k != 0)
        def _():
            h2_ref[0] += part

        @pl.when(jnp.logical_and(k == 7, c == 0))
        def _():
            al_ref[...] = jnp.dot(h2_ref[0], s_ref[0],
                                  preferred_element_type=_f32)

        @pl.when(jnp.logical_and(k == 7, c == 1))
        def _():
            al_ref[...] += jnp.dot(h2_ref[0], s_ref[0],
                                   preferred_element_type=_f32)

    return pl.pallas_call(
        body,
        grid=(10, 2, 8),
        in_specs=[
            pl.BlockSpec((1, 1024, 128), lambda nb, c, k: (k, nb, 0)),
            pl.BlockSpec((2, 1, 8, 128), lambda nb, c, k: (0, k, nb, 0)),
            pl.BlockSpec((1, 1, 128), lambda nb, c, k: (k, 0, 0)),
            pl.BlockSpec((1, 1, 128, 128), lambda nb, c, k: (c, k, 0, 0)),
            pl.BlockSpec((1, 128, 128), lambda nb, c, k: (c, 0, 0)),
        ],
        out_specs=[
            pl.BlockSpec((1, 1024, 128), lambda nb, c, k: (c, nb, 0)),
            pl.BlockSpec((1024, 128), lambda nb, c, k: (nb, 0)),
        ],
        out_shape=[
            jax.ShapeDtypeStruct((2, _NPAD, 128), _f32),
            jax.ShapeDtypeStruct((_NPAD, 128), _f32),
        ],
    )(msg1, den1, b1r, W2r, S2)


def _mm_c(msg2, den2, b2r):
    def body(m_ref, d_ref, b_ref, o_ref):
        den = d_ref[0, 0] + d_ref[1, 0] + 1e-16
        ov = m_ref[0].reshape(8, 128, 128) / den[:, :, None]
        o_ref[...] = ov.reshape(1024, 128) + b_ref[0, 0][None, :]

    return pl.pallas_call(
        body,
        grid=(10, 2),
        in_specs=[
            pl.BlockSpec((1, 1024, 128), lambda nb, c: (c, nb, 0)),
            pl.BlockSpec((2, 1, 8, 128), lambda nb, c: (0, 0, nb, 0)),
            pl.BlockSpec((1, 1, 128), lambda nb, c: (c, 0, 0)),
        ],
        out_specs=pl.BlockSpec((1024, 128), lambda nb, c: (nb, c)),
        out_shape=jax.ShapeDtypeStruct((_NPAD, _IN), _f32),
    )(msg2, den2, b2r)


def kernel(x, edge_index, W1, a1_src, a1_dst, b1, W2, a2_src, a2_dst, b2):
    S1 = jnp.zeros((8, 128, 128), _f32)
    for c in range(8):
        h = c // 2
        off = (c % 2) * 128
        S1 = S1.at[c, :, h].set(a1_src[h, off:off + 128])
        S1 = S1.at[c, :, _HEADS + h].set(a1_dst[h, off:off + 128])
    S2 = jnp.zeros((2, 128, 128), _f32)
    for c in range(2):
        S2 = S2.at[c, :, 0].set(a2_src[0, c * 128:(c + 1) * 128])
        S2 = S2.at[c, :, 1].set(a2_dst[0, c * 128:(c + 1) * 128])
    W2r = W2.reshape(8, 128, 2, 128).transpose(2, 0, 1, 3)
    b1r = b1.reshape(8, 1, 128)
    b2r = b2.reshape(2, 1, 128)
    x_pad = jnp.pad(x, ((0, _NPAD - _N), (0, 0)))

    pad = jnp.zeros((_EPAD - _E,), _i32)
    src_p = jnp.concatenate([edge_index[0], pad])
    dst_p = jnp.concatenate([edge_index[1], pad])
    dst2d = dst_p.reshape(_ROWS2D, _K)

    edge1 = _make_edge_kernel(_HEADS)
    edge2 = _make_edge_kernel(1)
    msg_k1 = _make_msg_kernel(8, _HEADS)
    msg_k2 = _make_msg_kernel(2, 1)

    h1, al1 = _mm_a(x_pad, W1, S1)
    alog1 = jnp.transpose(al1[:, :2 * _HEADS]).reshape(-1)
    p1, den1 = edge1(alog1, src_p, dst_p)
    msg1 = msg_k1(h1, src_p, dst2d, p1)

    den1k = den1.reshape(2, _HEADS, 80, 128)[:, jnp.array([0, 0, 1, 1, 2, 2, 3, 3])]
    h2, al2 = _mm_b(msg1, den1k, b1r, W2r, S2)

    alog2 = jnp.transpose(al2[:, :2]).reshape(-1)
    p2, den2 = edge2(alog2, src_p, dst_p)
    msg2 = msg_k2(h2, src_p, dst2d, p2)

    return _mm_c(msg2, den2.reshape(2, 1, 80, 128), b2r)[:_N]
